# Initial kernel scaffold; baseline (speedup 1.0000x reference)
import jax
import jax.numpy as jnp
from jax.experimental import pallas as pl


def kernel(x, edge_index, W_self, b_self, W_neigh, b_neigh):
    raise NotImplementedError("write your pallas kernel here")


# speedup vs baseline: 10.4347x; 10.4347x over previous
import functools

import jax
import jax.numpy as jnp
from jax import lax
from jax.experimental import pallas as pl
from jax.experimental.pallas import tpu as pltpu
from jax.experimental.pallas import tpu_sc as plsc

N_NODES = 10000
N_EDGES = 320000
D = 128

NC = 2
NS = 16
NW = NC * NS
CHUNK = 64
NBUF = 2
CPW = 160
NPHASE = 2
CPP = CPW // NPHASE
TOTAL_E = NW * CPW * CHUNK
ROWS_PER_TILE = 640
ACC_ROWS = NS * ROWS_PER_TILE

_mesh = plsc.VectorSubcoreMesh(core_axis_name="c", subcore_axis_name="s")


@functools.partial(
    pl.kernel,
    out_type=jax.ShapeDtypeStruct((NC, ACC_ROWS, D), jnp.float32),
    mesh=_mesh,
    scratch_types=[
        pltpu.VMEM((CPP, CHUNK), jnp.int32),
        pltpu.VMEM((CPP, CHUNK), jnp.int32),
        pltpu.VMEM((CHUNK, D), jnp.float32),
        pltpu.VMEM((CHUNK, D), jnp.float32),
        pltpu.VMEM_SHARED((ACC_ROWS, D), jnp.float32),
        pltpu.SemaphoreType.DMA,
        pltpu.SemaphoreType.DMA,
    ],
)
def _sc_aggregate(neigh_hbm, row_hbm, col_hbm, out_hbm,
                  row_v, col_v, buf0, buf1, acc, sem0, sem1):
    cid = lax.axis_index("c")
    sid = lax.axis_index("s")
    wid = cid * NS + sid

    zero16 = jnp.zeros((16,), jnp.float32)

    @pl.loop(0, CHUNK)
    def _zero_rows(r):
        for j in range(D // 16):
            buf0[r, pl.ds(j * 16, 16)] = zero16

    for t in range(ROWS_PER_TILE // CHUNK):
        pltpu.sync_copy(buf0, acc.at[pl.ds(sid * ROWS_PER_TILE + t * CHUNK, CHUNK)])

    plsc.subcore_barrier()

    bufs = (buf0, buf1)
    sems = (sem0, sem1)

    for phase in range(NPHASE):
        base = wid * CPW + phase * CPP
        pltpu.sync_copy(row_hbm.at[pl.ds(base, CPP)], row_v)
        pltpu.sync_copy(col_hbm.at[pl.ds(base, CPP)], col_v)

        pltpu.async_copy(neigh_hbm.at[row_v.at[0]], buf0, sem0)

        @pl.loop(0, CPP, step=NBUF)
        def _chunks(g):
            for b in range(NBUF):
                j = g + b
                @pl.when(j + 1 < CPP)
                def _():
                    pltpu.async_copy(
                        neigh_hbm.at[row_v.at[j + 1]], bufs[1 - b], sems[1 - b])
                pltpu.make_async_copy(
                    neigh_hbm.at[pl.ds(0, CHUNK)], bufs[b], sems[b]).wait()
                pltpu.sync_copy(bufs[b], acc.at[col_v.at[j]], add=True)

    plsc.subcore_barrier()

    pltpu.sync_copy(acc.at[pl.ds(sid * ROWS_PER_TILE, ROWS_PER_TILE)],
                    out_hbm.at[cid, pl.ds(sid * ROWS_PER_TILE, ROWS_PER_TILE)])


_BLK = 1000


def _mm_body(x_ref, wn_ref, bn_ref, ws_ref, bs_ref, neigh_ref, selfp_ref):
    xb = x_ref[...]
    dn = (((1,), (1,)), ((), ()))
    neigh_ref[...] = lax.dot_general(
        xb, wn_ref[...], dn, preferred_element_type=jnp.float32) + bn_ref[...]
    selfp_ref[...] = lax.dot_general(
        xb, ws_ref[...], dn, preferred_element_type=jnp.float32) + bs_ref[...]


def _addrelu_body(selfp_ref, p0_ref, p1_ref, o_ref):
    o_ref[...] = jnp.maximum(
        selfp_ref[...] + p0_ref[0, :, :] + p1_ref[0, :, :], 0.0)


def kernel(x, edge_index, W_self, b_self, W_neigh, b_neigh):
    ei = edge_index.astype(jnp.int32)
    row = ei[0]
    col = ei[1]
    npad = TOTAL_E - N_EDGES
    pad_iota = jnp.arange(npad, dtype=jnp.int32)
    pad_row = (pad_iota * 37) % N_NODES
    pad_col = N_NODES + pad_iota % (ACC_ROWS - N_NODES)
    row_p = jnp.concatenate([row, pad_row]).reshape(NW * CPW, CHUNK)
    col_p = jnp.concatenate([col, pad_col]).reshape(NW * CPW, CHUNK)

    nblk = N_NODES // _BLK
    neigh, selfp = pl.pallas_call(
        _mm_body,
        grid=(nblk,),
        in_specs=[
            pl.BlockSpec((_BLK, D), lambda i: (i, 0)),
            pl.BlockSpec((D, D), lambda i: (0, 0)),
            pl.BlockSpec((1, D), lambda i: (0, 0)),
            pl.BlockSpec((D, D), lambda i: (0, 0)),
            pl.BlockSpec((1, D), lambda i: (0, 0)),
        ],
        out_specs=[
            pl.BlockSpec((_BLK, D), lambda i: (i, 0)),
            pl.BlockSpec((_BLK, D), lambda i: (i, 0)),
        ],
        out_shape=[
            jax.ShapeDtypeStruct((N_NODES, D), jnp.float32),
            jax.ShapeDtypeStruct((N_NODES, D), jnp.float32),
        ],
    )(x, W_neigh, b_neigh.reshape(1, D), W_self, b_self.reshape(1, D))

    partials = _sc_aggregate(neigh, row_p, col_p)

    out = pl.pallas_call(
        _addrelu_body,
        grid=(nblk,),
        in_specs=[
            pl.BlockSpec((_BLK, D), lambda i: (i, 0)),
            pl.BlockSpec((1, _BLK, D), lambda i: (0, i, 0)),
            pl.BlockSpec((1, _BLK, D), lambda i: (1, i, 0)),
        ],
        out_specs=pl.BlockSpec((_BLK, D), lambda i: (i, 0)),
        out_shape=jax.ShapeDtypeStruct((N_NODES, D), jnp.float32),
    )(selfp, partials, partials)

    return out
